# Initial kernel scaffold; baseline (speedup 1.0000x reference)
import jax
import jax.numpy as jnp
from jax.experimental import pallas as pl


def kernel(x, edge_index, batch, W1, b1, W2, b2, W3, b3, Wm1, bm1, Wm2, bm2):
    raise NotImplementedError("write your pallas kernel here")


# speedup vs baseline: 13.5667x; 13.5667x over previous
import jax
import jax.numpy as jnp
from jax import lax
from jax.experimental import pallas as pl
from jax.experimental.pallas import tpu as pltpu
from jax.experimental.pallas import tpu_sc as plsc

NC = 2
NS = 16
CH = 125

_MESH = dict(core_axis_name="c", subcore_axis_name="s", num_cores=NC,
             num_subcores=NS)


def _sc_degree(dst2, n_pad):
    chunks, ch = dst2.shape
    cpt = chunks // NS
    rpt = n_pad // NS

    def body(dst_hbm, deg_hbm, idx_v, ones_v, accum_sh):
        c = lax.axis_index("c")
        s = lax.axis_index("s")
        for k in range(ones_v.shape[0] // 16):
            ones_v[pl.ds(k * 16, 16)] = jnp.full((16,), 1.0, jnp.float32)

        @pl.when(c == 0)
        def _():
            pltpu.sync_copy(dst_hbm.at[pl.ds(s * cpt, cpt)], idx_v)
            @pl.loop(0, rpt // 128)
            def _(r):
                pltpu.sync_copy(ones_v, accum_sh.at[pl.ds(s * rpt + r * 128, 128)])

        plsc.subcore_barrier()

        @pl.when(c == 0)
        def _():
            @pl.loop(0, cpt)
            def _(j):
                pltpu.sync_copy(ones_v.at[pl.ds(0, ch)],
                                accum_sh.at[idx_v.at[j]], add=True)

        plsc.subcore_barrier()

        @pl.when(c == 0)
        def _():
            pltpu.sync_copy(accum_sh.at[pl.ds(s * rpt, rpt)],
                            deg_hbm.at[pl.ds(s * rpt, rpt)])

    return pl.kernel(
        body,
        out_type=jax.ShapeDtypeStruct((n_pad,), jnp.float32),
        mesh=plsc.VectorSubcoreMesh(**_MESH),
        scratch_types=[
            pltpu.VMEM((cpt, ch), jnp.int32),
            pltpu.VMEM((128,), jnp.float32),
            pltpu.VMEM_SHARED((n_pad,), jnp.float32),
        ],
    )(dst2)


def _sc_aggregate(hw0, hw1, src2, dst2):
    chunks, ch = src2.shape
    cpt = chunks // NS
    n, d = hw0.shape
    rpt = n // NS

    gb = 16

    def body(hw0_hbm, hw1_hbm, src_hbm, dst_hbm, out0_hbm, out1_hbm,
             sidx, didx, buf, sem, accum_sh):
        c = lax.axis_index("c")
        s = lax.axis_index("s")
        sr = buf.shape[0]

        def run(hw_hbm, out_hbm):
            @pl.loop(0, rpt // sr)
            def _(r):
                pltpu.sync_copy(hw_hbm.at[pl.ds(s * rpt + r * sr, sr)], buf)
                pltpu.sync_copy(buf, accum_sh.at[pl.ds(s * rpt + r * sr, sr)])

            plsc.subcore_barrier()

            @pl.loop(0, cpt // gb)
            def _(g):
                pltpu.sync_copy(src_hbm.at[pl.ds(s * cpt + g * gb, gb)], sidx)
                pltpu.sync_copy(dst_hbm.at[pl.ds(s * cpt + g * gb, gb)], didx)

                @pl.loop(0, gb)
                def _(j):
                    pltpu.async_copy(hw_hbm.at[sidx.at[j]],
                                     buf.at[pl.ds(0, ch)], sem).wait()
                    pltpu.sync_copy(buf.at[pl.ds(0, ch)],
                                    accum_sh.at[didx.at[j]], add=True)

            plsc.subcore_barrier()

            @pl.loop(0, rpt // sr)
            def _(r):
                pltpu.sync_copy(accum_sh.at[pl.ds(s * rpt + r * sr, sr)], buf)
                pltpu.sync_copy(buf, out_hbm.at[pl.ds(s * rpt + r * sr, sr)])

        @pl.when(c == 0)
        def _():
            run(hw0_hbm, out0_hbm)

        @pl.when(c == 1)
        def _():
            run(hw1_hbm, out1_hbm)

    return pl.kernel(
        body,
        out_type=(jax.ShapeDtypeStruct((n, d), jnp.float32),
                  jax.ShapeDtypeStruct((n, d), jnp.float32)),
        mesh=plsc.VectorSubcoreMesh(**_MESH),
        scratch_types=[
            pltpu.VMEM((gb, ch), jnp.int32),
            pltpu.VMEM((gb, ch), jnp.int32),
            pltpu.VMEM((128, d), jnp.float32),
            pltpu.SemaphoreType.DMA,
            pltpu.VMEM_SHARED((n, d), jnp.float32),
        ],
    )(hw0, hw1, src2, dst2)


def _tc_layer1(x, W1, deg2, blk):
    n, d_in = x.shape
    d_h = W1.shape[1]
    hd = d_h // 2

    def body(x_ref, w_ref, deg_ref, hw0_ref, hw1_ref, dis_ref):
        dis = lax.rsqrt(deg_ref[...])
        y = jnp.dot(x_ref[...], w_ref[...],
                    preferred_element_type=jnp.float32) * dis
        hw0_ref[...] = y[:, :hd]
        hw1_ref[...] = y[:, hd:]
        dis_ref[...] = dis

    return pl.pallas_call(
        body,
        grid=(n // blk,),
        in_specs=[
            pl.BlockSpec((blk, d_in), lambda i: (i, 0)),
            pl.BlockSpec((d_in, d_h), lambda i: (0, 0)),
            pl.BlockSpec((blk, 1), lambda i: (i, 0)),
        ],
        out_specs=[
            pl.BlockSpec((blk, hd), lambda i: (i, 0)),
            pl.BlockSpec((blk, hd), lambda i: (i, 0)),
            pl.BlockSpec((blk, 1), lambda i: (i, 0)),
        ],
        out_shape=[
            jax.ShapeDtypeStruct((n, hd), jnp.float32),
            jax.ShapeDtypeStruct((n, hd), jnp.float32),
            jax.ShapeDtypeStruct((n, 1), jnp.float32),
        ],
    )(x, W1, deg2)


def _tc_mid(agg0, agg1, dis2, b_prev, W, blk):
    n, hd = agg0.shape
    d_h = W.shape[1]

    def body(a0_ref, a1_ref, dis_ref, b_ref, w_ref, hw0_ref, hw1_ref):
        dis = dis_ref[...]
        h = jnp.concatenate([a0_ref[...], a1_ref[...]], axis=1) * dis + b_ref[...]
        h = jnp.maximum(h, 0.0)
        y = jnp.dot(h, w_ref[...], preferred_element_type=jnp.float32) * dis
        hw0_ref[...] = y[:, :d_h // 2]
        hw1_ref[...] = y[:, d_h // 2:]

    return pl.pallas_call(
        body,
        grid=(n // blk,),
        in_specs=[
            pl.BlockSpec((blk, hd), lambda i: (i, 0)),
            pl.BlockSpec((blk, hd), lambda i: (i, 0)),
            pl.BlockSpec((blk, 1), lambda i: (i, 0)),
            pl.BlockSpec((1, 2 * hd), lambda i: (0, 0)),
            pl.BlockSpec((2 * hd, d_h), lambda i: (0, 0)),
        ],
        out_specs=[
            pl.BlockSpec((blk, d_h // 2), lambda i: (i, 0)),
            pl.BlockSpec((blk, d_h // 2), lambda i: (i, 0)),
        ],
        out_shape=[
            jax.ShapeDtypeStruct((n, d_h // 2), jnp.float32),
            jax.ShapeDtypeStruct((n, d_h // 2), jnp.float32),
        ],
    )(agg0, agg1, dis2, b_prev, W)


def _tc_pool_mlp(agg0, agg1, dis2, b3, batchf, Wm1, bm1, Wm2, bm2, n_graphs, blk):
    n, hd = agg0.shape
    d_h = 2 * hd
    d_out = Wm2.shape[1]
    nblk = n // blk

    def body(a0_ref, a1_ref, dis_ref, b_ref, bat_ref, wm1_ref, bm1_ref,
             wm2_ref, bm2_ref, out_ref, sums, cnts):
        i = pl.program_id(0)

        @pl.when(i == 0)
        def _():
            sums[...] = jnp.zeros_like(sums)
            cnts[...] = jnp.zeros_like(cnts)

        h3 = (jnp.concatenate([a0_ref[...], a1_ref[...]], axis=1)
              * dis_ref[...] + b_ref[...])
        ids = lax.broadcasted_iota(jnp.int32, (blk, n_graphs), 1).astype(jnp.float32)
        oh = (bat_ref[...] == ids).astype(jnp.float32)
        sums[...] += lax.dot_general(oh, h3, (((0,), (0,)), ((), ())),
                                     preferred_element_type=jnp.float32)
        cnts[...] += lax.dot_general(oh, jnp.ones((blk, 128), jnp.float32),
                                     (((0,), (0,)), ((), ())),
                                     preferred_element_type=jnp.float32)

        pooled = sums[...] / jnp.maximum(cnts[...][:, :1], 1.0)
        g = jnp.maximum(
            jnp.dot(pooled, wm1_ref[...], preferred_element_type=jnp.float32)
            + bm1_ref[...], 0.0)
        out_ref[...] = (jnp.dot(g, wm2_ref[...],
                                preferred_element_type=jnp.float32)
                        + bm2_ref[...])

    return pl.pallas_call(
        body,
        grid=(nblk,),
        in_specs=[
            pl.BlockSpec((blk, hd), lambda i: (i, 0)),
            pl.BlockSpec((blk, hd), lambda i: (i, 0)),
            pl.BlockSpec((blk, 1), lambda i: (i, 0)),
            pl.BlockSpec((1, d_h), lambda i: (0, 0)),
            pl.BlockSpec((blk, 1), lambda i: (i, 0)),
            pl.BlockSpec((d_h, d_h), lambda i: (0, 0)),
            pl.BlockSpec((1, d_h), lambda i: (0, 0)),
            pl.BlockSpec((d_h, d_out), lambda i: (0, 0)),
            pl.BlockSpec((1, d_out), lambda i: (0, 0)),
        ],
        out_specs=pl.BlockSpec((n_graphs, d_out), lambda i: (0, 0)),
        out_shape=jax.ShapeDtypeStruct((n_graphs, d_out), jnp.float32),
        scratch_shapes=[
            pltpu.VMEM((n_graphs, d_h), jnp.float32),
            pltpu.VMEM((n_graphs, 128), jnp.float32),
        ],
    )(agg0, agg1, dis2, b3, batchf, Wm1, bm1, Wm2, bm2)


def kernel(x, edge_index, batch, W1, b1, W2, b2, W3, b3, Wm1, bm1, Wm2, bm2):
    n = x.shape[0]
    e = edge_index.shape[1]
    n_graphs = 64

    ei = edge_index.astype(jnp.int32).reshape(2, e // CH, CH)
    src2 = ei[0]
    dst2 = ei[1]

    n_pad = ((n + NS * 128 - 1) // (NS * 128)) * (NS * 128)
    blk = 1024
    x = jnp.pad(x, ((0, n_pad - n), (0, 0)))
    batch = jnp.pad(batch.astype(jnp.int32), (0, n_pad - n),
                    constant_values=-1)

    deg = _sc_degree(dst2, n_pad)
    deg2 = deg.reshape(n_pad, 1)

    hw0, hw1, dis2 = _tc_layer1(x, W1, deg2, blk)
    a0, a1 = _sc_aggregate(hw0, hw1, src2, dst2)
    hw0, hw1 = _tc_mid(a0, a1, dis2, b1.reshape(1, -1), W2, blk)
    a0, a1 = _sc_aggregate(hw0, hw1, src2, dst2)
    hw0, hw1 = _tc_mid(a0, a1, dis2, b2.reshape(1, -1), W3, blk)
    a0, a1 = _sc_aggregate(hw0, hw1, src2, dst2)

    batchf = batch.astype(jnp.float32).reshape(n_pad, 1)
    out = _tc_pool_mlp(a0, a1, dis2, b3.reshape(1, -1), batchf,
                       Wm1, bm1.reshape(1, -1), Wm2, bm2.reshape(1, -1),
                       n_graphs, blk)
    return out


# speedup vs baseline: 19.9269x; 1.4688x over previous
import jax
import jax.numpy as jnp
from jax import lax
from jax.experimental import pallas as pl
from jax.experimental.pallas import tpu as pltpu
from jax.experimental.pallas import tpu_sc as plsc

NC = 2
NS = 16
CH = 125

_MESH = dict(core_axis_name="c", subcore_axis_name="s", num_cores=NC,
             num_subcores=NS)


def _sc_degree(dst2, n_pad):
    chunks, ch = dst2.shape
    cpt = chunks // NS
    rpt = n_pad // NS

    def body(dst_hbm, deg_hbm, idx_v, ones_v, accum_sh):
        c = lax.axis_index("c")
        s = lax.axis_index("s")
        for k in range(ones_v.shape[0] // 16):
            ones_v[pl.ds(k * 16, 16)] = jnp.full((16,), 1.0, jnp.float32)

        @pl.when(c == 0)
        def _():
            pltpu.sync_copy(dst_hbm.at[pl.ds(s * cpt, cpt)], idx_v)
            @pl.loop(0, rpt // 128)
            def _(r):
                pltpu.sync_copy(ones_v, accum_sh.at[pl.ds(s * rpt + r * 128, 128)])

        plsc.subcore_barrier()

        @pl.when(c == 0)
        def _():
            @pl.loop(0, cpt)
            def _(j):
                pltpu.sync_copy(ones_v.at[pl.ds(0, ch)],
                                accum_sh.at[idx_v.at[j]], add=True)

        plsc.subcore_barrier()

        @pl.when(c == 0)
        def _():
            pltpu.sync_copy(accum_sh.at[pl.ds(s * rpt, rpt)],
                            deg_hbm.at[pl.ds(s * rpt, rpt)])

    return pl.kernel(
        body,
        out_type=jax.ShapeDtypeStruct((n_pad,), jnp.float32),
        mesh=plsc.VectorSubcoreMesh(**_MESH),
        scratch_types=[
            pltpu.VMEM((cpt, ch), jnp.int32),
            pltpu.VMEM((128,), jnp.float32),
            pltpu.VMEM_SHARED((n_pad,), jnp.float32),
        ],
    )(dst2)


def _sc_aggregate(hw0, hw1, src2, dst2):
    chunks, ch = src2.shape
    cpt = chunks // NS
    n, d = hw0.shape
    rpt = n // NS

    gb = 16
    sr = 80

    def body(hw0_hbm, hw1_hbm, src_hbm, dst_hbm, out0_hbm, out1_hbm,
             sidx, didx, buf0, buf1, gs0, gs1, ss0, ss1, accum_sh):
        c = lax.axis_index("c")
        s = lax.axis_index("s")
        bufs = (buf0, buf1)
        gsems = (gs0, gs1)
        ssems = (ss0, ss1)

        def run(hw_hbm, out_hbm):
            @pl.loop(0, rpt // sr)
            def _(r):
                st = buf0.at[pl.ds(0, sr)]
                pltpu.sync_copy(hw_hbm.at[pl.ds(s * rpt + r * sr, sr)], st)
                pltpu.sync_copy(st, accum_sh.at[pl.ds(s * rpt + r * sr, sr)])

            plsc.subcore_barrier()

            @pl.loop(0, cpt // gb)
            def _(g):
                pltpu.sync_copy(src_hbm.at[pl.ds(s * cpt + g * gb, gb)], sidx)
                pltpu.sync_copy(dst_hbm.at[pl.ds(s * cpt + g * gb, gb)], didx)
                gd = [None] * gb
                sd = [None] * gb
                for p in range(gb):
                    if p >= 2:
                        sd[p - 2].wait()
                    gd[p] = pltpu.async_copy(hw_hbm.at[sidx.at[p]],
                                             bufs[p % 2], gsems[p % 2])
                    if p >= 1:
                        gd[p - 1].wait()
                        sd[p - 1] = pltpu.async_copy(
                            bufs[(p - 1) % 2], accum_sh.at[didx.at[p - 1]],
                            ssems[(p - 1) % 2], add=True)
                gd[gb - 1].wait()
                sd[gb - 1] = pltpu.async_copy(
                    bufs[(gb - 1) % 2], accum_sh.at[didx.at[gb - 1]],
                    ssems[(gb - 1) % 2], add=True)
                sd[gb - 2].wait()
                sd[gb - 1].wait()

            plsc.subcore_barrier()

            @pl.loop(0, rpt // sr)
            def _(r):
                st = buf0.at[pl.ds(0, sr)]
                pltpu.sync_copy(accum_sh.at[pl.ds(s * rpt + r * sr, sr)], st)
                pltpu.sync_copy(st, out_hbm.at[pl.ds(s * rpt + r * sr, sr)])

        @pl.when(c == 0)
        def _():
            run(hw0_hbm, out0_hbm)

        @pl.when(c == 1)
        def _():
            run(hw1_hbm, out1_hbm)

    return pl.kernel(
        body,
        out_type=(jax.ShapeDtypeStruct((n, d), jnp.float32),
                  jax.ShapeDtypeStruct((n, d), jnp.float32)),
        mesh=plsc.VectorSubcoreMesh(**_MESH),
        scratch_types=[
            pltpu.VMEM((gb, ch), jnp.int32),
            pltpu.VMEM((gb, ch), jnp.int32),
            pltpu.VMEM((ch, d), jnp.float32),
            pltpu.VMEM((ch, d), jnp.float32),
            pltpu.SemaphoreType.DMA,
            pltpu.SemaphoreType.DMA,
            pltpu.SemaphoreType.DMA,
            pltpu.SemaphoreType.DMA,
            pltpu.VMEM_SHARED((n, d), jnp.float32),
        ],
    )(hw0, hw1, src2, dst2)


def _tc_layer1(x, W1, deg2, blk):
    n, d_in = x.shape
    d_h = W1.shape[1]
    hd = d_h // 2

    def body(x_ref, w_ref, deg_ref, hw0_ref, hw1_ref, dis_ref):
        dis = lax.rsqrt(deg_ref[...])
        y = jnp.dot(x_ref[...], w_ref[...],
                    preferred_element_type=jnp.float32) * dis
        hw0_ref[...] = y[:, :hd]
        hw1_ref[...] = y[:, hd:]
        dis_ref[...] = dis

    return pl.pallas_call(
        body,
        grid=(n // blk,),
        in_specs=[
            pl.BlockSpec((blk, d_in), lambda i: (i, 0)),
            pl.BlockSpec((d_in, d_h), lambda i: (0, 0)),
            pl.BlockSpec((blk, 1), lambda i: (i, 0)),
        ],
        out_specs=[
            pl.BlockSpec((blk, hd), lambda i: (i, 0)),
            pl.BlockSpec((blk, hd), lambda i: (i, 0)),
            pl.BlockSpec((blk, 1), lambda i: (i, 0)),
        ],
        out_shape=[
            jax.ShapeDtypeStruct((n, hd), jnp.float32),
            jax.ShapeDtypeStruct((n, hd), jnp.float32),
            jax.ShapeDtypeStruct((n, 1), jnp.float32),
        ],
    )(x, W1, deg2)


def _tc_mid(agg0, agg1, dis2, b_prev, W, blk):
    n, hd = agg0.shape
    d_h = W.shape[1]

    def body(a0_ref, a1_ref, dis_ref, b_ref, w_ref, hw0_ref, hw1_ref):
        dis = dis_ref[...]
        h = jnp.concatenate([a0_ref[...], a1_ref[...]], axis=1) * dis + b_ref[...]
        h = jnp.maximum(h, 0.0)
        y = jnp.dot(h, w_ref[...], preferred_element_type=jnp.float32) * dis
        hw0_ref[...] = y[:, :d_h // 2]
        hw1_ref[...] = y[:, d_h // 2:]

    return pl.pallas_call(
        body,
        grid=(n // blk,),
        in_specs=[
            pl.BlockSpec((blk, hd), lambda i: (i, 0)),
            pl.BlockSpec((blk, hd), lambda i: (i, 0)),
            pl.BlockSpec((blk, 1), lambda i: (i, 0)),
            pl.BlockSpec((1, 2 * hd), lambda i: (0, 0)),
            pl.BlockSpec((2 * hd, d_h), lambda i: (0, 0)),
        ],
        out_specs=[
            pl.BlockSpec((blk, d_h // 2), lambda i: (i, 0)),
            pl.BlockSpec((blk, d_h // 2), lambda i: (i, 0)),
        ],
        out_shape=[
            jax.ShapeDtypeStruct((n, d_h // 2), jnp.float32),
            jax.ShapeDtypeStruct((n, d_h // 2), jnp.float32),
        ],
    )(agg0, agg1, dis2, b_prev, W)


def _tc_pool_mlp(agg0, agg1, dis2, b3, batchf, Wm1, bm1, Wm2, bm2, n_graphs, blk):
    n, hd = agg0.shape
    d_h = 2 * hd
    d_out = Wm2.shape[1]
    nblk = n // blk

    def body(a0_ref, a1_ref, dis_ref, b_ref, bat_ref, wm1_ref, bm1_ref,
             wm2_ref, bm2_ref, out_ref, sums, cnts):
        i = pl.program_id(0)

        @pl.when(i == 0)
        def _():
            sums[...] = jnp.zeros_like(sums)
            cnts[...] = jnp.zeros_like(cnts)

        h3 = (jnp.concatenate([a0_ref[...], a1_ref[...]], axis=1)
              * dis_ref[...] + b_ref[...])
        ids = lax.broadcasted_iota(jnp.int32, (blk, n_graphs), 1).astype(jnp.float32)
        oh = (bat_ref[...] == ids).astype(jnp.float32)
        sums[...] += lax.dot_general(oh, h3, (((0,), (0,)), ((), ())),
                                     preferred_element_type=jnp.float32)
        cnts[...] += lax.dot_general(oh, jnp.ones((blk, 128), jnp.float32),
                                     (((0,), (0,)), ((), ())),
                                     preferred_element_type=jnp.float32)

        pooled = sums[...] / jnp.maximum(cnts[...][:, :1], 1.0)
        g = jnp.maximum(
            jnp.dot(pooled, wm1_ref[...], preferred_element_type=jnp.float32)
            + bm1_ref[...], 0.0)
        out_ref[...] = (jnp.dot(g, wm2_ref[...],
                                preferred_element_type=jnp.float32)
                        + bm2_ref[...])

    return pl.pallas_call(
        body,
        grid=(nblk,),
        in_specs=[
            pl.BlockSpec((blk, hd), lambda i: (i, 0)),
            pl.BlockSpec((blk, hd), lambda i: (i, 0)),
            pl.BlockSpec((blk, 1), lambda i: (i, 0)),
            pl.BlockSpec((1, d_h), lambda i: (0, 0)),
            pl.BlockSpec((blk, 1), lambda i: (i, 0)),
            pl.BlockSpec((d_h, d_h), lambda i: (0, 0)),
            pl.BlockSpec((1, d_h), lambda i: (0, 0)),
            pl.BlockSpec((d_h, d_out), lambda i: (0, 0)),
            pl.BlockSpec((1, d_out), lambda i: (0, 0)),
        ],
        out_specs=pl.BlockSpec((n_graphs, d_out), lambda i: (0, 0)),
        out_shape=jax.ShapeDtypeStruct((n_graphs, d_out), jnp.float32),
        scratch_shapes=[
            pltpu.VMEM((n_graphs, d_h), jnp.float32),
            pltpu.VMEM((n_graphs, 128), jnp.float32),
        ],
    )(agg0, agg1, dis2, b3, batchf, Wm1, bm1, Wm2, bm2)


def kernel(x, edge_index, batch, W1, b1, W2, b2, W3, b3, Wm1, bm1, Wm2, bm2):
    n = x.shape[0]
    e = edge_index.shape[1]
    n_graphs = 64

    ei = edge_index.astype(jnp.int32).reshape(2, e // CH, CH)
    src2 = ei[0]
    dst2 = ei[1]

    n_pad = ((n + NS * 128 - 1) // (NS * 128)) * (NS * 128)
    blk = 1024
    x = jnp.pad(x, ((0, n_pad - n), (0, 0)))
    batch = jnp.pad(batch.astype(jnp.int32), (0, n_pad - n),
                    constant_values=-1)

    deg = _sc_degree(dst2, n_pad)
    deg2 = deg.reshape(n_pad, 1)

    hw0, hw1, dis2 = _tc_layer1(x, W1, deg2, blk)
    a0, a1 = _sc_aggregate(hw0, hw1, src2, dst2)
    hw0, hw1 = _tc_mid(a0, a1, dis2, b1.reshape(1, -1), W2, blk)
    a0, a1 = _sc_aggregate(hw0, hw1, src2, dst2)
    hw0, hw1 = _tc_mid(a0, a1, dis2, b2.reshape(1, -1), W3, blk)
    a0, a1 = _sc_aggregate(hw0, hw1, src2, dst2)

    batchf = batch.astype(jnp.float32).reshape(n_pad, 1)
    out = _tc_pool_mlp(a0, a1, dis2, b3.reshape(1, -1), batchf,
                       Wm1, bm1.reshape(1, -1), Wm2, bm2.reshape(1, -1),
                       n_graphs, blk)
    return out


# speedup vs baseline: 20.8175x; 1.0447x over previous
import jax
import jax.numpy as jnp
from jax import lax
from jax.experimental import pallas as pl
from jax.experimental.pallas import tpu as pltpu
from jax.experimental.pallas import tpu_sc as plsc

NC = 2
NS = 16
CH = 125

_MESH = dict(core_axis_name="c", subcore_axis_name="s", num_cores=NC,
             num_subcores=NS)


def _sc_degree(dst2, n_pad):
    chunks, ch = dst2.shape
    cpt = chunks // NS
    rpt = n_pad // NS

    def body(dst_hbm, deg_hbm, idx_v, ones_v, sem, accum_sh):
        c = lax.axis_index("c")
        s = lax.axis_index("s")
        for k in range(ones_v.shape[0] // 16):
            ones_v[pl.ds(k * 16, 16)] = jnp.full((16,), 1.0, jnp.float32)

        @pl.when(c == 0)
        def _():
            pltpu.sync_copy(dst_hbm.at[pl.ds(s * cpt, cpt)], idx_v)
            @pl.loop(0, rpt // 128)
            def _(r):
                pltpu.sync_copy(ones_v, accum_sh.at[pl.ds(s * rpt + r * 128, 128)])

        plsc.subcore_barrier()

        @pl.when(c == 0)
        def _():
            @pl.loop(0, cpt // 8)
            def _(g):
                ds_ = [pltpu.async_copy(ones_v.at[pl.ds(0, ch)],
                                        accum_sh.at[idx_v.at[g * 8 + p]],
                                        sem, add=True)
                       for p in range(8)]
                for dsc in ds_:
                    dsc.wait()

        plsc.subcore_barrier()

        @pl.when(c == 0)
        def _():
            pltpu.sync_copy(accum_sh.at[pl.ds(s * rpt, rpt)],
                            deg_hbm.at[pl.ds(s * rpt, rpt)])

    return pl.kernel(
        body,
        out_type=jax.ShapeDtypeStruct((n_pad,), jnp.float32),
        mesh=plsc.VectorSubcoreMesh(**_MESH),
        scratch_types=[
            pltpu.VMEM((cpt, ch), jnp.int32),
            pltpu.VMEM((128,), jnp.float32),
            pltpu.SemaphoreType.DMA,
            pltpu.VMEM_SHARED((n_pad,), jnp.float32),
        ],
    )(dst2)


def _sc_aggregate(hw0, hw1, src2, dst2):
    chunks, ch = src2.shape
    cpt = chunks // NS
    n, d = hw0.shape
    rpt = n // NS

    gb = 16
    sr = 80

    def body(hw0_hbm, hw1_hbm, src_hbm, dst_hbm, out0_hbm, out1_hbm,
             sidx, didx, buf0, buf1, gs0, gs1, ss0, ss1, is0, is1, accum_sh):
        c = lax.axis_index("c")
        s = lax.axis_index("s")
        bufs = (buf0, buf1)
        gsems = (gs0, gs1)
        ssems = (ss0, ss1)

        def run(hw_hbm, out_hbm):
            @pl.loop(0, rpt // sr)
            def _(r):
                st = buf0.at[pl.ds(0, sr)]
                pltpu.sync_copy(hw_hbm.at[pl.ds(s * rpt + r * sr, sr)], st)
                pltpu.sync_copy(st, accum_sh.at[pl.ds(s * rpt + r * sr, sr)])

            plsc.subcore_barrier()

            def pipeline(sidx_g, didx_g):
                gd = [None] * gb
                sd = [None] * gb
                for p in range(gb):
                    if p >= 2:
                        sd[p - 2].wait()
                    gd[p] = pltpu.async_copy(hw_hbm.at[sidx_g.at[p]],
                                             bufs[p % 2], gsems[p % 2])
                    if p >= 1:
                        gd[p - 1].wait()
                        sd[p - 1] = pltpu.async_copy(
                            bufs[(p - 1) % 2], accum_sh.at[didx_g.at[p - 1]],
                            ssems[(p - 1) % 2], add=True)
                gd[gb - 1].wait()
                sd[gb - 1] = pltpu.async_copy(
                    bufs[(gb - 1) % 2], accum_sh.at[didx_g.at[gb - 1]],
                    ssems[(gb - 1) % 2], add=True)
                sd[gb - 2].wait()
                sd[gb - 1].wait()

            def load_idx(g, sidx_g, didx_g, sem):
                a = pltpu.async_copy(
                    src_hbm.at[pl.ds(s * cpt + g * gb, gb)], sidx_g, sem)
                b = pltpu.async_copy(
                    dst_hbm.at[pl.ds(s * cpt + g * gb, gb)], didx_g, sem)
                return a, b

            npairs = cpt // (2 * gb)
            pa, pb = load_idx(0, sidx.at[0], didx.at[0], is0)
            pa.wait()
            pb.wait()

            @pl.loop(0, npairs)
            def _(k):
                na, nb = load_idx(2 * k + 1, sidx.at[1], didx.at[1], is1)
                pipeline(sidx.at[0], didx.at[0])
                na.wait()
                nb.wait()

                @pl.when(k < npairs - 1)
                def _():
                    la, lb = load_idx(2 * k + 2, sidx.at[0], didx.at[0], is0)
                    pipeline(sidx.at[1], didx.at[1])
                    la.wait()
                    lb.wait()

                @pl.when(k == npairs - 1)
                def _():
                    pipeline(sidx.at[1], didx.at[1])

            plsc.subcore_barrier()

            @pl.loop(0, rpt // sr)
            def _(r):
                st = buf0.at[pl.ds(0, sr)]
                pltpu.sync_copy(accum_sh.at[pl.ds(s * rpt + r * sr, sr)], st)
                pltpu.sync_copy(st, out_hbm.at[pl.ds(s * rpt + r * sr, sr)])

        @pl.when(c == 0)
        def _():
            run(hw0_hbm, out0_hbm)

        @pl.when(c == 1)
        def _():
            run(hw1_hbm, out1_hbm)

    return pl.kernel(
        body,
        out_type=(jax.ShapeDtypeStruct((n, d), jnp.float32),
                  jax.ShapeDtypeStruct((n, d), jnp.float32)),
        mesh=plsc.VectorSubcoreMesh(**_MESH),
        scratch_types=[
            pltpu.VMEM((2, gb, ch), jnp.int32),
            pltpu.VMEM((2, gb, ch), jnp.int32),
            pltpu.VMEM((ch, d), jnp.float32),
            pltpu.VMEM((ch, d), jnp.float32),
            pltpu.SemaphoreType.DMA,
            pltpu.SemaphoreType.DMA,
            pltpu.SemaphoreType.DMA,
            pltpu.SemaphoreType.DMA,
            pltpu.SemaphoreType.DMA,
            pltpu.SemaphoreType.DMA,
            pltpu.VMEM_SHARED((n, d), jnp.float32),
        ],
    )(hw0, hw1, src2, dst2)


def _tc_layer1(x, W1, deg2, blk):
    n, d_in = x.shape
    d_h = W1.shape[1]
    hd = d_h // 2

    def body(x_ref, w_ref, deg_ref, hw0_ref, hw1_ref, dis_ref):
        dis = lax.rsqrt(deg_ref[...])
        y = jnp.dot(x_ref[...], w_ref[...],
                    preferred_element_type=jnp.float32) * dis
        hw0_ref[...] = y[:, :hd]
        hw1_ref[...] = y[:, hd:]
        dis_ref[...] = dis

    return pl.pallas_call(
        body,
        grid=(n // blk,),
        in_specs=[
            pl.BlockSpec((blk, d_in), lambda i: (i, 0)),
            pl.BlockSpec((d_in, d_h), lambda i: (0, 0)),
            pl.BlockSpec((blk, 1), lambda i: (i, 0)),
        ],
        out_specs=[
            pl.BlockSpec((blk, hd), lambda i: (i, 0)),
            pl.BlockSpec((blk, hd), lambda i: (i, 0)),
            pl.BlockSpec((blk, 1), lambda i: (i, 0)),
        ],
        out_shape=[
            jax.ShapeDtypeStruct((n, hd), jnp.float32),
            jax.ShapeDtypeStruct((n, hd), jnp.float32),
            jax.ShapeDtypeStruct((n, 1), jnp.float32),
        ],
    )(x, W1, deg2)


def _tc_mid(agg0, agg1, dis2, b_prev, W, blk):
    n, hd = agg0.shape
    d_h = W.shape[1]

    def body(a0_ref, a1_ref, dis_ref, b_ref, w_ref, hw0_ref, hw1_ref):
        dis = dis_ref[...]
        h = jnp.concatenate([a0_ref[...], a1_ref[...]], axis=1) * dis + b_ref[...]
        h = jnp.maximum(h, 0.0)
        y = jnp.dot(h, w_ref[...], preferred_element_type=jnp.float32) * dis
        hw0_ref[...] = y[:, :d_h // 2]
        hw1_ref[...] = y[:, d_h // 2:]

    return pl.pallas_call(
        body,
        grid=(n // blk,),
        in_specs=[
            pl.BlockSpec((blk, hd), lambda i: (i, 0)),
            pl.BlockSpec((blk, hd), lambda i: (i, 0)),
            pl.BlockSpec((blk, 1), lambda i: (i, 0)),
            pl.BlockSpec((1, 2 * hd), lambda i: (0, 0)),
            pl.BlockSpec((2 * hd, d_h), lambda i: (0, 0)),
        ],
        out_specs=[
            pl.BlockSpec((blk, d_h // 2), lambda i: (i, 0)),
            pl.BlockSpec((blk, d_h // 2), lambda i: (i, 0)),
        ],
        out_shape=[
            jax.ShapeDtypeStruct((n, d_h // 2), jnp.float32),
            jax.ShapeDtypeStruct((n, d_h // 2), jnp.float32),
        ],
    )(agg0, agg1, dis2, b_prev, W)


def _tc_pool_mlp(agg0, agg1, dis2, b3, batchf, Wm1, bm1, Wm2, bm2, n_graphs, blk):
    n, hd = agg0.shape
    d_h = 2 * hd
    d_out = Wm2.shape[1]
    nblk = n // blk

    def body(a0_ref, a1_ref, dis_ref, b_ref, bat_ref, wm1_ref, bm1_ref,
             wm2_ref, bm2_ref, out_ref, sums, cnts):
        i = pl.program_id(0)

        @pl.when(i == 0)
        def _():
            sums[...] = jnp.zeros_like(sums)
            cnts[...] = jnp.zeros_like(cnts)

        h3 = (jnp.concatenate([a0_ref[...], a1_ref[...]], axis=1)
              * dis_ref[...] + b_ref[...])
        ids = lax.broadcasted_iota(jnp.int32, (blk, n_graphs), 1).astype(jnp.float32)
        oh = (bat_ref[...] == ids).astype(jnp.float32)
        sums[...] += lax.dot_general(oh, h3, (((0,), (0,)), ((), ())),
                                     preferred_element_type=jnp.float32)
        cnts[...] += lax.dot_general(oh, jnp.ones((blk, 128), jnp.float32),
                                     (((0,), (0,)), ((), ())),
                                     preferred_element_type=jnp.float32)

        pooled = sums[...] / jnp.maximum(cnts[...][:, :1], 1.0)
        g = jnp.maximum(
            jnp.dot(pooled, wm1_ref[...], preferred_element_type=jnp.float32)
            + bm1_ref[...], 0.0)
        out_ref[...] = (jnp.dot(g, wm2_ref[...],
                                preferred_element_type=jnp.float32)
                        + bm2_ref[...])

    return pl.pallas_call(
        body,
        grid=(nblk,),
        in_specs=[
            pl.BlockSpec((blk, hd), lambda i: (i, 0)),
            pl.BlockSpec((blk, hd), lambda i: (i, 0)),
            pl.BlockSpec((blk, 1), lambda i: (i, 0)),
            pl.BlockSpec((1, d_h), lambda i: (0, 0)),
            pl.BlockSpec((blk, 1), lambda i: (i, 0)),
            pl.BlockSpec((d_h, d_h), lambda i: (0, 0)),
            pl.BlockSpec((1, d_h), lambda i: (0, 0)),
            pl.BlockSpec((d_h, d_out), lambda i: (0, 0)),
            pl.BlockSpec((1, d_out), lambda i: (0, 0)),
        ],
        out_specs=pl.BlockSpec((n_graphs, d_out), lambda i: (0, 0)),
        out_shape=jax.ShapeDtypeStruct((n_graphs, d_out), jnp.float32),
        scratch_shapes=[
            pltpu.VMEM((n_graphs, d_h), jnp.float32),
            pltpu.VMEM((n_graphs, 128), jnp.float32),
        ],
    )(agg0, agg1, dis2, b3, batchf, Wm1, bm1, Wm2, bm2)


def kernel(x, edge_index, batch, W1, b1, W2, b2, W3, b3, Wm1, bm1, Wm2, bm2):
    n = x.shape[0]
    e = edge_index.shape[1]
    n_graphs = 64

    ei = edge_index.astype(jnp.int32).reshape(2, e // CH, CH)
    src2 = ei[0]
    dst2 = ei[1]

    n_pad = ((n + NS * 128 - 1) // (NS * 128)) * (NS * 128)
    blk = 1024
    x = jnp.pad(x, ((0, n_pad - n), (0, 0)))
    batch = jnp.pad(batch.astype(jnp.int32), (0, n_pad - n),
                    constant_values=-1)

    deg = _sc_degree(dst2, n_pad)
    deg2 = deg.reshape(n_pad, 1)

    hw0, hw1, dis2 = _tc_layer1(x, W1, deg2, blk)
    a0, a1 = _sc_aggregate(hw0, hw1, src2, dst2)
    hw0, hw1 = _tc_mid(a0, a1, dis2, b1.reshape(1, -1), W2, blk)
    a0, a1 = _sc_aggregate(hw0, hw1, src2, dst2)
    hw0, hw1 = _tc_mid(a0, a1, dis2, b2.reshape(1, -1), W3, blk)
    a0, a1 = _sc_aggregate(hw0, hw1, src2, dst2)

    batchf = batch.astype(jnp.float32).reshape(n_pad, 1)
    out = _tc_pool_mlp(a0, a1, dis2, b3.reshape(1, -1), batchf,
                       Wm1, bm1.reshape(1, -1), Wm2, bm2.reshape(1, -1),
                       n_graphs, blk)
    return out


# speedup vs baseline: 21.3699x; 1.0265x over previous
import jax
import jax.numpy as jnp
from jax import lax
from jax.experimental import pallas as pl
from jax.experimental.pallas import tpu as pltpu
from jax.experimental.pallas import tpu_sc as plsc

NC = 2
NS = 16
CH = 125

_MESH = dict(core_axis_name="c", subcore_axis_name="s", num_cores=NC,
             num_subcores=NS)


def _sc_degree(dst2, n_pad):
    chunks, ch = dst2.shape
    cpt = chunks // NS
    rpt = n_pad // NS

    def body(dst_hbm, deg_hbm, idx_v, ones_v, sem, accum_sh):
        c = lax.axis_index("c")
        s = lax.axis_index("s")
        for k in range(ones_v.shape[0] // 16):
            ones_v[pl.ds(k * 16, 16)] = jnp.full((16,), 1.0, jnp.float32)

        @pl.when(c == 0)
        def _():
            pltpu.sync_copy(dst_hbm.at[pl.ds(s * cpt, cpt)], idx_v)
            @pl.loop(0, rpt // 128)
            def _(r):
                pltpu.sync_copy(ones_v, accum_sh.at[pl.ds(s * rpt + r * 128, 128)])

        plsc.subcore_barrier()

        @pl.when(c == 0)
        def _():
            @pl.loop(0, cpt // 8)
            def _(g):
                ds_ = [pltpu.async_copy(ones_v.at[pl.ds(0, ch)],
                                        accum_sh.at[idx_v.at[g * 8 + p]],
                                        sem, add=True)
                       for p in range(8)]
                for dsc in ds_:
                    dsc.wait()

        plsc.subcore_barrier()

        @pl.when(c == 0)
        def _():
            pltpu.sync_copy(accum_sh.at[pl.ds(s * rpt, rpt)],
                            deg_hbm.at[pl.ds(s * rpt, rpt)])

    return pl.kernel(
        body,
        out_type=jax.ShapeDtypeStruct((n_pad,), jnp.float32),
        mesh=plsc.VectorSubcoreMesh(**_MESH),
        scratch_types=[
            pltpu.VMEM((cpt, ch), jnp.int32),
            pltpu.VMEM((128,), jnp.float32),
            pltpu.SemaphoreType.DMA,
            pltpu.VMEM_SHARED((n_pad,), jnp.float32),
        ],
    )(dst2)


def _sc_aggregate(hw0, hw1, src2, dst2):
    chunks, ch = src2.shape
    cpt = chunks // NS
    n, d = hw0.shape
    rpt = n // NS

    gb = 16

    def body(hw0_hbm, hw1_hbm, src_hbm, dst_hbm, out0_hbm, out1_hbm,
             sidx, didx, buf0, buf1, gs0, gs1, ss0, ss1, is0, is1, accum_sh):
        c = lax.axis_index("c")
        s = lax.axis_index("s")
        bufs = (buf0, buf1)
        gsems = (gs0, gs1)
        ssems = (ss0, ss1)

        def run(hw_hbm, out_hbm):
            pltpu.sync_copy(hw_hbm.at[pl.ds(s * rpt, rpt)],
                            accum_sh.at[pl.ds(s * rpt, rpt)])

            plsc.subcore_barrier()

            def pipeline(sidx_g, didx_g):
                gd = [None] * gb
                sd = [None] * gb
                for p in range(gb):
                    if p >= 2:
                        sd[p - 2].wait()
                    gd[p] = pltpu.async_copy(hw_hbm.at[sidx_g.at[p]],
                                             bufs[p % 2], gsems[p % 2])
                    if p >= 1:
                        gd[p - 1].wait()
                        sd[p - 1] = pltpu.async_copy(
                            bufs[(p - 1) % 2], accum_sh.at[didx_g.at[p - 1]],
                            ssems[(p - 1) % 2], add=True)
                gd[gb - 1].wait()
                sd[gb - 1] = pltpu.async_copy(
                    bufs[(gb - 1) % 2], accum_sh.at[didx_g.at[gb - 1]],
                    ssems[(gb - 1) % 2], add=True)
                sd[gb - 2].wait()
                sd[gb - 1].wait()

            def load_idx(g, sidx_g, didx_g, sem):
                a = pltpu.async_copy(
                    src_hbm.at[pl.ds(s * cpt + g * gb, gb)], sidx_g, sem)
                b = pltpu.async_copy(
                    dst_hbm.at[pl.ds(s * cpt + g * gb, gb)], didx_g, sem)
                return a, b

            npairs = cpt // (2 * gb)
            pa, pb = load_idx(0, sidx.at[0], didx.at[0], is0)
            pa.wait()
            pb.wait()

            @pl.loop(0, npairs)
            def _(k):
                na, nb = load_idx(2 * k + 1, sidx.at[1], didx.at[1], is1)
                pipeline(sidx.at[0], didx.at[0])
                na.wait()
                nb.wait()

                @pl.when(k < npairs - 1)
                def _():
                    la, lb = load_idx(2 * k + 2, sidx.at[0], didx.at[0], is0)
                    pipeline(sidx.at[1], didx.at[1])
                    la.wait()
                    lb.wait()

                @pl.when(k == npairs - 1)
                def _():
                    pipeline(sidx.at[1], didx.at[1])

            plsc.subcore_barrier()

            pltpu.sync_copy(accum_sh.at[pl.ds(s * rpt, rpt)],
                            out_hbm.at[pl.ds(s * rpt, rpt)])

        @pl.when(c == 0)
        def _():
            run(hw0_hbm, out0_hbm)

        @pl.when(c == 1)
        def _():
            run(hw1_hbm, out1_hbm)

    return pl.kernel(
        body,
        out_type=(jax.ShapeDtypeStruct((n, d), jnp.float32),
                  jax.ShapeDtypeStruct((n, d), jnp.float32)),
        mesh=plsc.VectorSubcoreMesh(**_MESH),
        scratch_types=[
            pltpu.VMEM((2, gb, ch), jnp.int32),
            pltpu.VMEM((2, gb, ch), jnp.int32),
            pltpu.VMEM((ch, d), jnp.float32),
            pltpu.VMEM((ch, d), jnp.float32),
            pltpu.SemaphoreType.DMA,
            pltpu.SemaphoreType.DMA,
            pltpu.SemaphoreType.DMA,
            pltpu.SemaphoreType.DMA,
            pltpu.SemaphoreType.DMA,
            pltpu.SemaphoreType.DMA,
            pltpu.VMEM_SHARED((n, d), jnp.float32),
        ],
    )(hw0, hw1, src2, dst2)


def _tc_layer1(x, W1, deg2, blk):
    n, d_in = x.shape
    d_h = W1.shape[1]
    hd = d_h // 2

    def body(x_ref, w_ref, deg_ref, hw0_ref, hw1_ref, dis_ref):
        dis = lax.rsqrt(deg_ref[...])
        y = jnp.dot(x_ref[...], w_ref[...],
                    preferred_element_type=jnp.float32) * dis
        hw0_ref[...] = y[:, :hd]
        hw1_ref[...] = y[:, hd:]
        dis_ref[...] = dis

    return pl.pallas_call(
        body,
        grid=(n // blk,),
        in_specs=[
            pl.BlockSpec((blk, d_in), lambda i: (i, 0)),
            pl.BlockSpec((d_in, d_h), lambda i: (0, 0)),
            pl.BlockSpec((blk, 1), lambda i: (i, 0)),
        ],
        out_specs=[
            pl.BlockSpec((blk, hd), lambda i: (i, 0)),
            pl.BlockSpec((blk, hd), lambda i: (i, 0)),
            pl.BlockSpec((blk, 1), lambda i: (i, 0)),
        ],
        out_shape=[
            jax.ShapeDtypeStruct((n, hd), jnp.float32),
            jax.ShapeDtypeStruct((n, hd), jnp.float32),
            jax.ShapeDtypeStruct((n, 1), jnp.float32),
        ],
    )(x, W1, deg2)


def _tc_mid(agg0, agg1, dis2, b_prev, W, blk):
    n, hd = agg0.shape
    d_h = W.shape[1]

    def body(a0_ref, a1_ref, dis_ref, b_ref, w_ref, hw0_ref, hw1_ref):
        dis = dis_ref[...]
        h = jnp.concatenate([a0_ref[...], a1_ref[...]], axis=1) * dis + b_ref[...]
        h = jnp.maximum(h, 0.0)
        y = jnp.dot(h, w_ref[...], preferred_element_type=jnp.float32) * dis
        hw0_ref[...] = y[:, :d_h // 2]
        hw1_ref[...] = y[:, d_h // 2:]

    return pl.pallas_call(
        body,
        grid=(n // blk,),
        in_specs=[
            pl.BlockSpec((blk, hd), lambda i: (i, 0)),
            pl.BlockSpec((blk, hd), lambda i: (i, 0)),
            pl.BlockSpec((blk, 1), lambda i: (i, 0)),
            pl.BlockSpec((1, 2 * hd), lambda i: (0, 0)),
            pl.BlockSpec((2 * hd, d_h), lambda i: (0, 0)),
        ],
        out_specs=[
            pl.BlockSpec((blk, d_h // 2), lambda i: (i, 0)),
            pl.BlockSpec((blk, d_h // 2), lambda i: (i, 0)),
        ],
        out_shape=[
            jax.ShapeDtypeStruct((n, d_h // 2), jnp.float32),
            jax.ShapeDtypeStruct((n, d_h // 2), jnp.float32),
        ],
    )(agg0, agg1, dis2, b_prev, W)


def _tc_pool_mlp(agg0, agg1, dis2, b3, batchf, Wm1, bm1, Wm2, bm2, n_graphs, blk):
    n, hd = agg0.shape
    d_h = 2 * hd
    d_out = Wm2.shape[1]
    nblk = n // blk

    def body(a0_ref, a1_ref, dis_ref, b_ref, bat_ref, wm1_ref, bm1_ref,
             wm2_ref, bm2_ref, out_ref, sums, cnts):
        i = pl.program_id(0)

        @pl.when(i == 0)
        def _():
            sums[...] = jnp.zeros_like(sums)
            cnts[...] = jnp.zeros_like(cnts)

        h3 = (jnp.concatenate([a0_ref[...], a1_ref[...]], axis=1)
              * dis_ref[...] + b_ref[...])
        ids = lax.broadcasted_iota(jnp.int32, (blk, n_graphs), 1).astype(jnp.float32)
        oh = (bat_ref[...] == ids).astype(jnp.float32)
        sums[...] += lax.dot_general(oh, h3, (((0,), (0,)), ((), ())),
                                     preferred_element_type=jnp.float32)
        cnts[...] += lax.dot_general(oh, jnp.ones((blk, 128), jnp.float32),
                                     (((0,), (0,)), ((), ())),
                                     preferred_element_type=jnp.float32)

        pooled = sums[...] / jnp.maximum(cnts[...][:, :1], 1.0)
        g = jnp.maximum(
            jnp.dot(pooled, wm1_ref[...], preferred_element_type=jnp.float32)
            + bm1_ref[...], 0.0)
        out_ref[...] = (jnp.dot(g, wm2_ref[...],
                                preferred_element_type=jnp.float32)
                        + bm2_ref[...])

    return pl.pallas_call(
        body,
        grid=(nblk,),
        in_specs=[
            pl.BlockSpec((blk, hd), lambda i: (i, 0)),
            pl.BlockSpec((blk, hd), lambda i: (i, 0)),
            pl.BlockSpec((blk, 1), lambda i: (i, 0)),
            pl.BlockSpec((1, d_h), lambda i: (0, 0)),
            pl.BlockSpec((blk, 1), lambda i: (i, 0)),
            pl.BlockSpec((d_h, d_h), lambda i: (0, 0)),
            pl.BlockSpec((1, d_h), lambda i: (0, 0)),
            pl.BlockSpec((d_h, d_out), lambda i: (0, 0)),
            pl.BlockSpec((1, d_out), lambda i: (0, 0)),
        ],
        out_specs=pl.BlockSpec((n_graphs, d_out), lambda i: (0, 0)),
        out_shape=jax.ShapeDtypeStruct((n_graphs, d_out), jnp.float32),
        scratch_shapes=[
            pltpu.VMEM((n_graphs, d_h), jnp.float32),
            pltpu.VMEM((n_graphs, 128), jnp.float32),
        ],
    )(agg0, agg1, dis2, b3, batchf, Wm1, bm1, Wm2, bm2)


def kernel(x, edge_index, batch, W1, b1, W2, b2, W3, b3, Wm1, bm1, Wm2, bm2):
    n = x.shape[0]
    e = edge_index.shape[1]
    n_graphs = 64

    ei = edge_index.astype(jnp.int32).reshape(2, e // CH, CH)
    src2 = ei[0]
    dst2 = ei[1]

    n_pad = ((n + NS * 128 - 1) // (NS * 128)) * (NS * 128)
    blk = 1024
    x = jnp.pad(x, ((0, n_pad - n), (0, 0)))
    batch = jnp.pad(batch.astype(jnp.int32), (0, n_pad - n),
                    constant_values=-1)

    deg = _sc_degree(dst2, n_pad)
    deg2 = deg.reshape(n_pad, 1)

    hw0, hw1, dis2 = _tc_layer1(x, W1, deg2, blk)
    a0, a1 = _sc_aggregate(hw0, hw1, src2, dst2)
    hw0, hw1 = _tc_mid(a0, a1, dis2, b1.reshape(1, -1), W2, blk)
    a0, a1 = _sc_aggregate(hw0, hw1, src2, dst2)
    hw0, hw1 = _tc_mid(a0, a1, dis2, b2.reshape(1, -1), W3, blk)
    a0, a1 = _sc_aggregate(hw0, hw1, src2, dst2)

    batchf = batch.astype(jnp.float32).reshape(n_pad, 1)
    out = _tc_pool_mlp(a0, a1, dis2, b3.reshape(1, -1), batchf,
                       Wm1, bm1.reshape(1, -1), Wm2, bm2.reshape(1, -1),
                       n_graphs, blk)
    return out
